# baseline (speedup 1.0000x reference)
import functools

import jax
import jax.numpy as jnp
import numpy as np
from jax import lax
from jax.experimental import pallas as pl
from jax.experimental.pallas import tpu as pltpu
from jax.experimental.pallas import tpu_sc as plsc

B = 32
V = 1_000_000
VPAD = 1 << 20
C = 16384
NC = VPAD // C
CUTOFF = np.float32(0.9)

_I32 = jnp.int32
_TINY = np.float32(np.finfo(np.float32).tiny)


def _rotl(x, r):
  return lax.shift_left(x, np.int32(r)) | lax.shift_right_logical(
      x, np.int32(32 - r))


def _threefry_bits(x1):
  ks0 = np.int32(0)
  ks1 = np.int32(42)
  ks2 = np.int32(np.uint32(0 ^ 42 ^ 0x1BD11BDA).astype(np.int32))
  r0 = (13, 15, 26, 6)
  r1 = (17, 29, 16, 24)

  x0 = jnp.full_like(x1, ks0)
  x1 = x1 + ks1

  def rounds(x0, x1, rots):
    for r in rots:
      x0 = x0 + x1
      x1 = _rotl(x1, r)
      x1 = x0 ^ x1
    return x0, x1

  x0, x1 = rounds(x0, x1, r0)
  x0, x1 = x0 + ks1, x1 + ks2 + np.int32(1)
  x0, x1 = rounds(x0, x1, r1)
  x0, x1 = x0 + ks2, x1 + ks0 + np.int32(2)
  x0, x1 = rounds(x0, x1, r0)
  x0, x1 = x0 + ks0, x1 + ks1 + np.int32(3)
  x0, x1 = rounds(x0, x1, r1)
  x0, x1 = x0 + ks1, x1 + ks2 + np.int32(4)
  x0, x1 = rounds(x0, x1, r0)
  x0, x1 = x0 + ks2, x1 + ks0 + np.int32(5)
  return x0 ^ x1


def _gumbel(flat_idx_i32):
  bits = _threefry_bits(flat_idx_i32)
  fb = lax.shift_right_logical(bits, np.int32(9)) | np.int32(0x3F800000)
  f = lax.bitcast_convert_type(fb, jnp.float32) - np.float32(1.0)
  u = jnp.maximum(_TINY, f + _TINY)
  return -jnp.log(-jnp.log(u))


def _sample_body(sv_ref, out_ref, state):
  r = pl.program_id(0)
  p = pl.program_id(1)
  c = pl.program_id(2)

  rows = C // 128
  jj = (c * C + jax.lax.broadcasted_iota(_I32, (rows, 128), 0) * 128
        + jax.lax.broadcasted_iota(_I32, (rows, 128), 1))
  valid = jj < V
  v = jnp.where(valid, sv_ref[0], np.float32(0.0))

  x = v
  ri = jax.lax.broadcasted_iota(_I32, (rows, rows), 0)
  ci = jax.lax.broadcasted_iota(_I32, (rows, rows), 1)
  lstrict = (ri > ci).astype(jnp.float32)
  li = jax.lax.broadcasted_iota(_I32, (128, 128), 0)
  lj = jax.lax.broadcasted_iota(_I32, (128, 128), 1)
  ut = (li <= lj).astype(jnp.float32)
  lanecum = jax.lax.dot_general(
      x, ut, (((1,), (0,)), ((), ())), preferred_element_type=jnp.float32)
  sub = jax.lax.dot_general(
      lstrict, x, (((1,), (0,)), ((), ())), preferred_element_type=jnp.float32)
  cum_in = lanecum + jnp.sum(sub, axis=1, keepdims=True)

  @pl.when((p == 0) & (c == 0))
  def _():
    state[0] = np.float32(0.0)
    state[1] = np.float32(0.0)

  @pl.when(p == 0)
  def _():
    carry = state[0]
    cum = cum_in + carry
    kept = cum <= CUTOFF
    state[1] = state[1] + jnp.sum(jnp.where(kept, v, np.float32(0.0)))
    state[0] = carry + jnp.sum(v)

  @pl.when((p == 1) & (c == 0))
  def _():
    state[2] = np.float32(-np.inf)
    state[3] = np.float32(2.0e9)
    state[0] = np.float32(0.0)

  @pl.when(p == 1)
  def _():
    carry = state[0]
    total = jnp.maximum(state[1], np.float32(1e-10))
    cum = cum_in + carry
    kept = cum <= CUTOFF
    w = jnp.where(kept, v / total, np.float32(0.0))
    flat = r * np.int32(V) + jj
    g = _gumbel(flat)
    s = jnp.log(w + np.float32(1e-10)) + g
    s = jnp.where(valid, s, np.float32(-np.inf))
    m = jnp.max(s)
    jl = jnp.min(jnp.where(s == m, jj, np.int32(2**31 - 1))).astype(jnp.float32)
    best = state[2]
    bestj = state[3]
    better = (m > best) | ((m == best) & (jl < bestj))
    state[2] = jnp.where(better, m, best)
    state[3] = jnp.where(better, jl, bestj)
    state[0] = carry + jnp.sum(v)

  @pl.when((p == 1) & (c == NC - 1))
  def _():
    out_ref[0, r] = state[3].astype(_I32)


def _sample_rank(svals):
  sv3 = svals.reshape(B * NC, C // 128, 128)
  out = pl.pallas_call(
      _sample_body,
      grid=(B, 2, NC),
      in_specs=[
          pl.BlockSpec((1, C // 128, 128), lambda r, p, c: (r * NC + c, 0, 0))
      ],
      out_specs=pl.BlockSpec(memory_space=pltpu.SMEM),
      out_shape=jax.ShapeDtypeStruct((1, B), _I32),
      scratch_shapes=[pltpu.SMEM((8,), jnp.float32)],
  )(sv3)
  return out[0]


W = 20000
NWIN = V // W
NV = W // 16
RADIX = 2048
NHV = RADIX // 16


def _digit(k, shift):
  return lax.shift_right_logical(k, np.int32(shift)) & np.int32(0x7FF)


def _sc_sort_body(keys_hbm, skeys, sidx, bka, bia, bkb, bib,
                  kbuf, ibuf, pos, ha, hb, sem):
  row = lax.axis_index("s") * 2 + lax.axis_index("c")
  rbase = row * V
  rbase_pad = row * VPAD

  ones = jnp.ones((16,), _I32)
  zeros = jnp.zeros((16,), _I32)
  iota = lax.iota(_I32, 16)

  cnt_cal, _ = plsc.scan_count(zeros)
  cntbase = jnp.min(cnt_cal)
  cs_cal = plsc.cumsum(ones)
  csoff = np.int32(1) - jnp.min(cs_cal)

  def zero_hist(h):
    def zb(i, _):
      h[pl.ds(i * 16, 16)] = zeros
      return 0
    lax.fori_loop(0, NHV, zb, 0)

  def prefix_hist(h):
    def pb(i, carry):
      hv = h[pl.ds(i * 16, 16)]
      incl = plsc.cumsum(hv) + csoff * hv
      h[pl.ds(i * 16, 16)] = incl - hv + carry
      return carry + jnp.sum(hv)
    lax.fori_loop(0, NHV, pb, np.int32(0))

  zero_hist(ha)

  def hist0_win(win, _):
    pltpu.sync_copy(keys_hbm.at[pl.ds(rbase + win * W, W)], kbuf)
    def body(t, _):
      k = kbuf[pl.ds(t * 16, 16)]
      plsc.addupdate_scatter(ha, [_digit(k, 0)], ones)
      return 0
    lax.fori_loop(0, NV, body, 0)
    return 0
  lax.fori_loop(0, NWIN, hist0_win, 0)

  def permute_pass(shift, src_k, src_i, dst_k, dst_i, hcur, hnxt, nshift,
                   dst_base):
    if hnxt is not None:
      zero_hist(hnxt)

    def win_body(win, _):
      pltpu.sync_copy(src_k.at[pl.ds(rbase + win * W, W)], kbuf)
      if src_i is not None:
        pltpu.sync_copy(src_i.at[pl.ds(rbase + win * W, W)], ibuf)

      def body(t, _):
        k = kbuf[pl.ds(t * 16, 16)]
        d = _digit(k, shift)
        base = plsc.load_gather(hcur, [d])
        cnt, last = plsc.scan_count(d)
        pv = base + (cnt - cntbase)
        pos[pl.ds(t * 16, 16)] = pv + dst_base
        plsc.store_scatter(hcur, [d], pv + 1, mask=last)
        if hnxt is not None:
          plsc.addupdate_scatter(hnxt, [_digit(k, nshift)], ones)
        if src_i is None:
          ibuf[pl.ds(t * 16, 16)] = win * W + t * 16 + iota
        return 0
      lax.fori_loop(0, NV, body, 0)

      cp1 = pltpu.async_copy(kbuf, dst_k.at[pos], sem)
      cp2 = pltpu.async_copy(ibuf, dst_i.at[pos], sem)
      cp1.wait()
      cp2.wait()
      return 0
    lax.fori_loop(0, NWIN, win_body, 0)

  prefix_hist(ha)
  permute_pass(0, keys_hbm, None, bka, bia, ha, hb, 11, rbase)
  prefix_hist(hb)
  permute_pass(11, bka, bia, bkb, bib, hb, ha, 22, rbase)
  prefix_hist(ha)
  permute_pass(22, bkb, bib, skeys, sidx, ha, None, 0, rbase_pad)


@jax.jit
def _sc_sort(keys):
  mesh = plsc.VectorSubcoreMesh(core_axis_name="c", subcore_axis_name="s")
  f = pl.kernel(
      _sc_sort_body,
      out_type=[
          jax.ShapeDtypeStruct((B * VPAD,), _I32),
          jax.ShapeDtypeStruct((B * VPAD,), _I32),
          jax.ShapeDtypeStruct((B * V,), _I32),
          jax.ShapeDtypeStruct((B * V,), _I32),
          jax.ShapeDtypeStruct((B * V,), _I32),
          jax.ShapeDtypeStruct((B * V,), _I32),
      ],
      mesh=mesh,
      compiler_params=pltpu.CompilerParams(needs_layout_passes=False),
      scratch_types=[
          pltpu.VMEM((W,), _I32),
          pltpu.VMEM((W,), _I32),
          pltpu.VMEM((W,), _I32),
          pltpu.VMEM((RADIX,), _I32),
          pltpu.VMEM((RADIX,), _I32),
          pltpu.SemaphoreType.DMA,
      ],
  )
  skeys, sidx, *_ = f(keys.reshape(B * V))
  return skeys.reshape(B, VPAD), sidx.reshape(B, VPAD)


def _sort_descending(probs):
  keys = ~lax.bitcast_convert_type(probs, _I32)
  skeys, sidx = _sc_sort(keys)
  svals = lax.bitcast_convert_type(~skeys, jnp.float32)
  return svals, sidx


def kernel(logits):
  probs = jax.nn.softmax(logits, axis=-1)
  svals, sidx = _sort_descending(probs)
  jstar = _sample_rank(svals)
  tok = jnp.take_along_axis(sidx, jstar[:, None], axis=-1)[:, 0]
  return tok
